# Initial kernel scaffold; baseline (speedup 1.0000x reference)
import jax
import jax.numpy as jnp
from jax.experimental import pallas as pl


def kernel(x, emb):
    raise NotImplementedError("write your pallas kernel here")


# speedup vs baseline: 1.1029x; 1.1029x over previous
import functools

import jax
import jax.numpy as jnp
from jax import lax
from jax.experimental import pallas as pl
from jax.experimental.pallas import tpu as pltpu
from jax.experimental.pallas import tpu_sc as plsc

_N_E = 8192
_E_DIM = 256
_B = 16384
_BETA = 0.25
_BM = 256
_NB = _B // _BM
_EPS = 1e-12


def _argmin_body(x_ref, emb_ref, idx_ref, cbn_ref):
    @pl.when(pl.program_id(0) == 0)
    def _():
        e = emb_ref[...]
        n = jnp.sqrt(jnp.sum(e * e, axis=1, keepdims=True))
        cbn_ref[...] = e / jnp.maximum(n, _EPS)

    x = x_ref[...]
    xn = x / jnp.maximum(jnp.sqrt(jnp.sum(x * x, axis=1, keepdims=True)), _EPS)
    sim = lax.dot_general(xn, cbn_ref[...], (((1,), (1,)), ((), ())),
                          preferred_element_type=jnp.float32)
    d = 1.0 - sim
    dmin = jnp.min(d, axis=1, keepdims=True)
    col = lax.broadcasted_iota(jnp.int32, d.shape, 1)
    idx = jnp.min(jnp.where(d == dmin, col, _N_E), axis=1)
    idx_ref[0, 0, :] = idx.astype(jnp.int32)


def _argmin_call(x, emb):
    return pl.pallas_call(
        _argmin_body,
        grid=(_NB,),
        in_specs=[
            pl.BlockSpec((_BM, _E_DIM), lambda i: (i, 0)),
            pl.BlockSpec((_N_E, _E_DIM), lambda i: (0, 0)),
        ],
        out_specs=pl.BlockSpec((1, 1, _BM), lambda i: (i, 0, 0)),
        out_shape=jax.ShapeDtypeStruct((_NB, 1, _BM), jnp.int32),
        scratch_shapes=[pltpu.VMEM((_N_E, _E_DIM), jnp.float32)],
    )(x, emb)


_NCORES = 2
_NSUB = 16
_NW = _NCORES * _NSUB
_CH = 128
_NCH = _B // (_NW * _CH)


@functools.cache
def _make_gather_sc():
    @functools.partial(
        pl.kernel,
        mesh=plsc.VectorSubcoreMesh(core_axis_name="c", subcore_axis_name="s"),
        out_type=jax.ShapeDtypeStruct((_B, _E_DIM), jnp.float32),
        scratch_types=[
            pltpu.VMEM((_CH,), jnp.int32),
            pltpu.VMEM((_CH, _E_DIM), jnp.float32),
            pltpu.SemaphoreType.DMA,
        ],
    )
    def _gather_sc(emb_hbm, idx_hbm, out_hbm, idx_v, rows_v, sem):
        wid = lax.axis_index("s") * _NCORES + lax.axis_index("c")
        for j in range(_NCH):
            pltpu.sync_copy(idx_hbm.at[wid, j], idx_v)
            pltpu.async_copy(emb_hbm.at[idx_v], rows_v, sem).wait()
            pltpu.sync_copy(rows_v,
                            out_hbm.at[pl.ds((wid * _NCH + j) * _CH, _CH)])

    return _gather_sc


def _proj_body(x_ref, cv_ref, xq_ref, sc_ref, loss_ref, acc_ref):
    @pl.when(pl.program_id(0) == 0)
    def _():
        acc_ref[0, 0] = 0.0

    x = x_ref[...]
    cv = cv_ref[...]
    dot = jnp.sum(x * cv, axis=1, keepdims=True)
    nsq = jnp.sum(cv * cv, axis=1, keepdims=True)
    scalar = dot / (nsq + 1e-08)
    proj = scalar * cv
    xq_ref[...] = x + (proj - x)
    sc_ref[0, 0, :] = scalar[:, 0]
    acc_ref[0, 0] += jnp.sum((proj - x) ** 2)

    @pl.when(pl.program_id(0) == _NB - 1)
    def _():
        m = acc_ref[0, 0] / (_B * _E_DIM)
        loss_ref[...] = jnp.reshape(m + _BETA * m, (1, 1))


def _proj_call(x, cv):
    return pl.pallas_call(
        _proj_body,
        grid=(_NB,),
        in_specs=[
            pl.BlockSpec((_BM, _E_DIM), lambda i: (i, 0)),
            pl.BlockSpec((_BM, _E_DIM), lambda i: (i, 0)),
        ],
        out_specs=[
            pl.BlockSpec((_BM, _E_DIM), lambda i: (i, 0)),
            pl.BlockSpec((1, 1, _BM), lambda i: (i, 0, 0)),
            pl.BlockSpec((1, 1), lambda i: (0, 0)),
        ],
        out_shape=[
            jax.ShapeDtypeStruct((_B, _E_DIM), jnp.float32),
            jax.ShapeDtypeStruct((_NB, 1, _BM), jnp.float32),
            jax.ShapeDtypeStruct((1, 1), jnp.float32),
        ],
        scratch_shapes=[pltpu.SMEM((1, 1), jnp.float32)],
    )(x, cv)


def kernel(x, emb):
    idx3 = _argmin_call(x, emb)
    indices = idx3.reshape(_B)
    cv = _make_gather_sc()(emb, indices.reshape(_NW, _NCH, _CH))
    xq, sc3, loss11 = _proj_call(x, cv)
    return (xq, loss11[0, 0], indices, sc3.reshape(_B))


# speedup vs baseline: 1.3511x; 1.2250x over previous
import functools

import jax
import jax.numpy as jnp
from jax import lax
from jax.experimental import pallas as pl
from jax.experimental.pallas import tpu as pltpu
from jax.experimental.pallas import tpu_sc as plsc

_N_E = 8192
_E_DIM = 256
_B = 16384
_BETA = 0.25
_BM = 256
_NB = _B // _BM
_EPS = 1e-12


def _argmin_body(x_ref, emb_ref, idx_ref, cbn_ref):
    @pl.when(pl.program_id(0) == 0)
    def _():
        e = emb_ref[...]
        n = jnp.sqrt(jnp.sum(e * e, axis=1, keepdims=True))
        cbn_ref[...] = e / jnp.maximum(n, _EPS)

    x = x_ref[...]
    xn = x / jnp.maximum(jnp.sqrt(jnp.sum(x * x, axis=1, keepdims=True)), _EPS)
    sim = lax.dot_general(xn, cbn_ref[...], (((1,), (1,)), ((), ())),
                          preferred_element_type=jnp.float32)
    _C = 128
    cur = 1.0 - sim[:, 0:_C]
    cur_j = jnp.zeros((_BM, _C), jnp.int32)
    for j in range(1, _N_E // _C):
        dj = 1.0 - sim[:, j * _C:(j + 1) * _C]
        lt = dj < cur
        cur = jnp.where(lt, dj, cur)
        cur_j = jnp.where(lt, j, cur_j)
    lane = lax.broadcasted_iota(jnp.int32, (_BM, _C), 1)
    col = cur_j * _C + lane
    m = jnp.min(cur, axis=1, keepdims=True)
    idx = jnp.min(jnp.where(cur == m, col, _N_E), axis=1)
    idx_ref[0, 0, :] = idx.astype(jnp.int32)


def _argmin_call(x, emb):
    return pl.pallas_call(
        _argmin_body,
        grid=(_NB,),
        in_specs=[
            pl.BlockSpec((_BM, _E_DIM), lambda i: (i, 0)),
            pl.BlockSpec((_N_E, _E_DIM), lambda i: (0, 0)),
        ],
        out_specs=pl.BlockSpec((1, 1, _BM), lambda i: (i, 0, 0)),
        out_shape=jax.ShapeDtypeStruct((_NB, 1, _BM), jnp.int32),
        scratch_shapes=[pltpu.VMEM((_N_E, _E_DIM), jnp.float32)],
    )(x, emb)


_NCORES = 2
_NSUB = 16
_NW = _NCORES * _NSUB
_CH = 128
_NCH = _B // (_NW * _CH)


@functools.cache
def _make_gather_sc():
    @functools.partial(
        pl.kernel,
        mesh=plsc.VectorSubcoreMesh(core_axis_name="c", subcore_axis_name="s"),
        out_type=jax.ShapeDtypeStruct((_B, _E_DIM), jnp.float32),
        scratch_types=[
            pltpu.VMEM((_CH,), jnp.int32),
            pltpu.VMEM((_CH, _E_DIM), jnp.float32),
            pltpu.SemaphoreType.DMA,
        ],
    )
    def _gather_sc(emb_hbm, idx_hbm, out_hbm, idx_v, rows_v, sem):
        wid = lax.axis_index("s") * _NCORES + lax.axis_index("c")
        for j in range(_NCH):
            pltpu.sync_copy(idx_hbm.at[wid, j], idx_v)
            pltpu.async_copy(emb_hbm.at[idx_v], rows_v, sem).wait()
            pltpu.sync_copy(rows_v,
                            out_hbm.at[pl.ds((wid * _NCH + j) * _CH, _CH)])

    return _gather_sc


def _proj_body(x_ref, cv_ref, xq_ref, sc_ref, loss_ref, acc_ref):
    @pl.when(pl.program_id(0) == 0)
    def _():
        acc_ref[0, 0] = 0.0

    x = x_ref[...]
    cv = cv_ref[...]
    dot = jnp.sum(x * cv, axis=1, keepdims=True)
    nsq = jnp.sum(cv * cv, axis=1, keepdims=True)
    scalar = dot / (nsq + 1e-08)
    proj = scalar * cv
    xq_ref[...] = x + (proj - x)
    sc_ref[0, 0, :] = scalar[:, 0]
    acc_ref[0, 0] += jnp.sum((proj - x) ** 2)

    @pl.when(pl.program_id(0) == _NB - 1)
    def _():
        m = acc_ref[0, 0] / (_B * _E_DIM)
        loss_ref[...] = jnp.reshape(m + _BETA * m, (1, 1))


def _proj_call(x, cv):
    return pl.pallas_call(
        _proj_body,
        grid=(_NB,),
        in_specs=[
            pl.BlockSpec((_BM, _E_DIM), lambda i: (i, 0)),
            pl.BlockSpec((_BM, _E_DIM), lambda i: (i, 0)),
        ],
        out_specs=[
            pl.BlockSpec((_BM, _E_DIM), lambda i: (i, 0)),
            pl.BlockSpec((1, 1, _BM), lambda i: (i, 0, 0)),
            pl.BlockSpec((1, 1), lambda i: (0, 0)),
        ],
        out_shape=[
            jax.ShapeDtypeStruct((_B, _E_DIM), jnp.float32),
            jax.ShapeDtypeStruct((_NB, 1, _BM), jnp.float32),
            jax.ShapeDtypeStruct((1, 1), jnp.float32),
        ],
        scratch_shapes=[pltpu.SMEM((1, 1), jnp.float32)],
    )(x, cv)


def kernel(x, emb):
    idx3 = _argmin_call(x, emb)
    indices = idx3.reshape(_B)
    cv = _make_gather_sc()(emb, indices.reshape(_NW, _NCH, _CH))
    xq, sc3, loss11 = _proj_call(x, cv)
    return (xq, loss11[0, 0], indices, sc3.reshape(_B))
